# baseline (speedup 1.0000x reference)
import functools

import jax
import jax.numpy as jnp
from jax.experimental import pallas as pl
from jax.experimental.pallas import tpu as pltpu

E = 64
DIM = 768
HID = 2048
TOKENS = 2048
TT = 128
NT = TOKENS // TT


def _gating_kernel(x_ref, gw_ref, wall_ref):
    x = x_ref[...]
    gw = gw_ref[...]
    logits = jax.lax.dot_general(
        x, gw, (((1,), (1,)), ((), ())), preferred_element_type=jnp.float32
    )
    s = jax.nn.softmax(logits, axis=-1)
    e_iota = jax.lax.broadcasted_iota(jnp.int32, s.shape, 1)
    v1 = jnp.max(s, axis=-1)
    i1 = jnp.argmax(s, axis=-1)
    s2 = jnp.where(e_iota == i1[:, None], -jnp.inf, s)
    v2 = jnp.max(s2, axis=-1)
    i2 = jnp.argmax(s2, axis=-1)
    tot = v1 + v2 + 1e-20
    wa = (v1 / tot)[:, None]
    wb = (v2 / tot)[:, None]
    wall = jnp.where(e_iota == i1[:, None], wa, 0.0) + jnp.where(
        e_iota == i2[:, None], wb, 0.0
    )
    wall_ref[...] = wall


def _ffn_kernel(wall_ref, x_ref, w1_ref, w2_ref, w3_ref, out_ref):
    e = pl.program_id(0)
    t = pl.program_id(1)
    x = x_ref[...]
    w1 = w1_ref[0]
    w3 = w3_ref[0]
    w2 = w2_ref[0]
    g = jax.lax.dot_general(
        x, w1, (((1,), (1,)), ((), ())), preferred_element_type=jnp.float32
    )
    u = jax.lax.dot_general(
        x, w3, (((1,), (1,)), ((), ())), preferred_element_type=jnp.float32
    )
    h = (g * jax.nn.sigmoid(g)) * u
    o = jax.lax.dot_general(
        h, w2, (((1,), (1,)), ((), ())), preferred_element_type=jnp.float32
    )
    wall = wall_ref[...]
    e_iota = jax.lax.broadcasted_iota(jnp.int32, wall.shape, 1)
    w = jnp.sum(jnp.where(e_iota == e, wall, 0.0), axis=1, keepdims=True)
    contrib = o * w

    @pl.when(e == 0)
    def _init():
        out_ref[pl.ds(t * TT, TT), :] = contrib

    @pl.when(e > 0)
    def _acc():
        out_ref[pl.ds(t * TT, TT), :] += contrib


@jax.jit
def kernel(x, gate_weight, w1, w2, w3):
    b, s, d = x.shape
    xf = x.reshape(-1, d)

    wall = pl.pallas_call(
        _gating_kernel,
        grid=(NT,),
        in_specs=[
            pl.BlockSpec((TT, DIM), lambda t: (t, 0)),
            pl.BlockSpec((E, DIM), lambda t: (0, 0)),
        ],
        out_specs=pl.BlockSpec((TT, E), lambda t: (t, 0)),
        out_shape=jax.ShapeDtypeStruct((TOKENS, E), jnp.float32),
    )(xf, gate_weight)

    out = pl.pallas_call(
        _ffn_kernel,
        grid=(E, NT),
        in_specs=[
            pl.BlockSpec((TT, E), lambda e, t: (t, 0)),
            pl.BlockSpec((TT, DIM), lambda e, t: (t, 0)),
            pl.BlockSpec((1, HID, DIM), lambda e, t: (e, 0, 0)),
            pl.BlockSpec((1, DIM, HID), lambda e, t: (e, 0, 0)),
            pl.BlockSpec((1, HID, DIM), lambda e, t: (e, 0, 0)),
        ],
        out_specs=pl.BlockSpec((TOKENS, DIM), lambda e, t: (0, 0)),
        out_shape=jax.ShapeDtypeStruct((TOKENS, DIM), jnp.float32),
        compiler_params=pltpu.CompilerParams(
            dimension_semantics=("arbitrary", "arbitrary"),
        ),
    )(wall, xf, w1, w2, w3)

    return out.reshape(b, s, d)


# speedup vs baseline: 9.2220x; 9.2220x over previous
import functools

import jax
import jax.numpy as jnp
from jax.experimental import pallas as pl
from jax.experimental.pallas import tpu as pltpu

E = 64
DIM = 768
HID = 2048
TOKENS = 2048
TT = 128
GMAX = 96
PADN = GMAX * TT


def _routing_kernel(x_ref, gw_ref, te_ref, nt_ref, pos1_ref, pos2_ref,
                    w1n_ref, w2n_ref):
    xf = x_ref[...]
    gw = gw_ref[...]
    logits = jax.lax.dot_general(
        xf, gw, (((1,), (1,)), ((), ())), preferred_element_type=jnp.float32
    )
    s = jax.nn.softmax(logits, axis=-1)
    e_iota = jax.lax.broadcasted_iota(jnp.int32, s.shape, 1)
    v1 = jnp.max(s, axis=-1)
    i1 = jnp.argmax(s, axis=-1)
    s2 = jnp.where(e_iota == i1[:, None], -jnp.inf, s)
    v2 = jnp.max(s2, axis=-1)
    i2 = jnp.argmax(s2, axis=-1)
    tot = v1 + v2 + 1e-20
    w1n_ref[...] = v1 / tot
    w2n_ref[...] = v2 / tot

    o1 = (e_iota == i1[:, None]).astype(jnp.float32)
    o2 = (e_iota == i2[:, None]).astype(jnp.float32)
    c = o1 + o2

    r_iota = jax.lax.broadcasted_iota(jnp.int32, (TOKENS, TOKENS), 0)
    c_iota = jax.lax.broadcasted_iota(jnp.int32, (TOKENS, TOKENS), 1)
    ltri = (c_iota < r_iota).astype(jnp.float32)
    excl = jax.lax.dot_general(
        ltri, c, (((1,), (0,)), ((), ())), preferred_element_type=jnp.float32
    )

    counts = jnp.sum(c, axis=0, keepdims=True)
    ptiles = jnp.floor((counts + (TT - 1)) / TT)
    ee_r = jax.lax.broadcasted_iota(jnp.int32, (E, E), 0)
    ee_c = jax.lax.broadcasted_iota(jnp.int32, (E, E), 1)
    mstrict = (ee_r < ee_c).astype(jnp.float32)
    cum_excl = jax.lax.dot_general(
        ptiles, mstrict, (((1,), (0,)), ((), ())),
        preferred_element_type=jnp.float32,
    )
    po = cum_excl * TT

    base = po + excl
    pos1 = jnp.sum(base * o1, axis=1)
    pos2 = jnp.sum(base * o2, axis=1)
    pos1_ref[...] = pos1.astype(jnp.int32)
    pos2_ref[...] = pos2.astype(jnp.int32)

    cum_incl = cum_excl + ptiles
    j_iota = jax.lax.broadcasted_iota(jnp.int32, (GMAX, E), 0)
    te = jnp.sum((cum_incl.astype(jnp.int32) <= j_iota).astype(jnp.int32),
                 axis=1)
    te_ref[...] = jnp.minimum(te, E - 1)
    nt_ref[...] = jnp.sum(ptiles, axis=1).astype(jnp.int32)


def _moe_kernel(te_ref, nt_ref, pos1_ref, pos2_ref, w1n_ref, w2n_ref,
                x_ref, w1_ref, w2_ref, w3_ref, out_ref):
    j = pl.program_id(0)
    n = nt_ref[0]

    @pl.when(j < n)
    def _work():
        slots = j * TT + jax.lax.broadcasted_iota(jnp.int32, (TT, 1), 0)
        cmp1 = (pos1_ref[...][None, :] == slots).astype(jnp.float32)
        cmp2 = (pos2_ref[...][None, :] == slots).astype(jnp.float32)
        gather = cmp1 + cmp2
        xs = jax.lax.dot_general(
            gather, x_ref[...], (((1,), (0,)), ((), ())),
            preferred_element_type=jnp.float32,
        )
        w1 = w1_ref[0]
        w3 = w3_ref[0]
        w2 = w2_ref[0]
        g = jax.lax.dot_general(
            xs, w1, (((1,), (1,)), ((), ())),
            preferred_element_type=jnp.float32,
        )
        u = jax.lax.dot_general(
            xs, w3, (((1,), (1,)), ((), ())),
            preferred_element_type=jnp.float32,
        )
        h = (g * jax.nn.sigmoid(g)) * u
        o = jax.lax.dot_general(
            h, w2, (((1,), (1,)), ((), ())),
            preferred_element_type=jnp.float32,
        )
        sw = (w1n_ref[...][None, :] * cmp1 +
              w2n_ref[...][None, :] * cmp2)
        contrib = jax.lax.dot_general(
            sw, o, (((0,), (0,)), ((), ())),
            preferred_element_type=jnp.float32,
        )

        @pl.when(j == 0)
        def _init():
            out_ref[...] = contrib

        @pl.when(j > 0)
        def _acc():
            out_ref[...] += contrib


@jax.jit
def kernel(x, gate_weight, w1, w2, w3):
    b, s, d = x.shape
    xf = x.reshape(-1, d)

    te, nt, pos1, pos2, w1n, w2n = pl.pallas_call(
        _routing_kernel,
        grid=(1,),
        in_specs=[
            pl.BlockSpec((TOKENS, DIM), lambda i: (0, 0)),
            pl.BlockSpec((E, DIM), lambda i: (0, 0)),
        ],
        out_specs=[
            pl.BlockSpec((GMAX,), lambda i: (0,)),
            pl.BlockSpec((1,), lambda i: (0,)),
            pl.BlockSpec((TOKENS,), lambda i: (0,)),
            pl.BlockSpec((TOKENS,), lambda i: (0,)),
            pl.BlockSpec((TOKENS,), lambda i: (0,)),
            pl.BlockSpec((TOKENS,), lambda i: (0,)),
        ],
        out_shape=[
            jax.ShapeDtypeStruct((GMAX,), jnp.int32),
            jax.ShapeDtypeStruct((1,), jnp.int32),
            jax.ShapeDtypeStruct((TOKENS,), jnp.int32),
            jax.ShapeDtypeStruct((TOKENS,), jnp.int32),
            jax.ShapeDtypeStruct((TOKENS,), jnp.float32),
            jax.ShapeDtypeStruct((TOKENS,), jnp.float32),
        ],
    )(xf, gate_weight)

    grid_spec = pltpu.PrefetchScalarGridSpec(
        num_scalar_prefetch=2,
        grid=(GMAX,),
        in_specs=[
            pl.BlockSpec((TOKENS,), lambda j, te, nt: (0,)),
            pl.BlockSpec((TOKENS,), lambda j, te, nt: (0,)),
            pl.BlockSpec((TOKENS,), lambda j, te, nt: (0,)),
            pl.BlockSpec((TOKENS,), lambda j, te, nt: (0,)),
            pl.BlockSpec((TOKENS, DIM), lambda j, te, nt: (0, 0)),
            pl.BlockSpec((1, HID, DIM), lambda j, te, nt: (te[j], 0, 0)),
            pl.BlockSpec((1, DIM, HID), lambda j, te, nt: (te[j], 0, 0)),
            pl.BlockSpec((1, HID, DIM), lambda j, te, nt: (te[j], 0, 0)),
        ],
        out_specs=pl.BlockSpec((TOKENS, DIM), lambda j, te, nt: (0, 0)),
    )
    out = pl.pallas_call(
        _moe_kernel,
        grid_spec=grid_spec,
        out_shape=jax.ShapeDtypeStruct((TOKENS, DIM), jnp.float32),
        compiler_params=pltpu.CompilerParams(
            dimension_semantics=("arbitrary",),
        ),
    )(te, nt, pos1, pos2, w1n, w2n, xf, w1, w2, w3)

    return out.reshape(b, s, d)
